# Initial kernel scaffold; baseline (speedup 1.0000x reference)
import jax
import jax.numpy as jnp
from jax.experimental import pallas as pl


def kernel(scores, bbox_deltas, im_info):
    raise NotImplementedError("write your pallas kernel here")


# speedup vs baseline: 6.8111x; 6.8111x over previous
import jax
import jax.numpy as jnp
from jax import lax
from jax.experimental import pallas as pl
from jax.experimental.pallas import tpu as pltpu

A = 9
H = 50
W = 50
HW = H * W
N = A * HW
RPAD = 176
NP = RPAD * 128
PRE = 20
POST = 10
NMS_T = 0.7
NEG = -1e9
NEGINF = -3.0e38


def _body(im_ref, fg_ref, dx_ref, dy_ref, dw_ref, dh_ref, out_ref,
          x1s, y1s, x2s, y2s, fgs):
    shape = (RPAD, 128)
    Fi = (lax.broadcasted_iota(jnp.int32, shape, 0) * 128
          + lax.broadcasted_iota(jnp.int32, shape, 1))
    F = Fi.astype(jnp.float32)
    p = jnp.floor(F / 9.0)
    a = F - p * 9.0
    hh = jnp.floor(p / 50.0)
    ww = p - hh * 50.0
    ridx = jnp.floor(a / 3.0)
    sidx = a - ridx * 3.0
    wr = jnp.where(ridx == 0.0, 23.0, jnp.where(ridx == 1.0, 16.0, 11.0))
    hr = jnp.where(ridx == 0.0, 12.0, jnp.where(ridx == 1.0, 16.0, 22.0))
    sc = jnp.where(sidx == 0.0, 8.0, jnp.where(sidx == 1.0, 16.0, 32.0))
    wa = wr * sc
    ha = hr * sc
    cx = 8.0 + ww * 16.0
    cy = 8.0 + hh * 16.0

    imh = im_ref[0, 0]
    imw = im_ref[0, 1]
    imsc = im_ref[0, 2]

    dx = dx_ref[...]
    dy = dy_ref[...]
    dw = jnp.clip(dw_ref[...], -10.0, 4.135)
    dh = jnp.clip(dh_ref[...], -10.0, 4.135)
    pcx = dx * wa + cx
    pcy = dy * ha + cy
    pw = jnp.exp(dw) * wa
    ph = jnp.exp(dh) * ha
    x1 = jnp.clip(pcx - 0.5 * pw, 0.0, imw - 1.0)
    x2 = jnp.clip(pcx + 0.5 * pw, 0.0, imw - 1.0)
    y1 = jnp.clip(pcy - 0.5 * ph, 0.0, imh - 1.0)
    y2 = jnp.clip(pcy + 0.5 * ph, 0.0, imh - 1.0)
    ws_ = x2 - x1 + 1.0
    hs_ = y2 - y1 + 1.0
    minsz = 5.0 * imsc
    valid = (ws_ >= minsz) & (hs_ >= minsz)
    fgm = jnp.where(valid, fg_ref[...], NEG)
    fgm = jnp.where(Fi < N, fgm, NEGINF)

    x1s[...] = x1
    y1s[...] = y1
    x2s[...] = x2
    y2s[...] = y2
    fgs[...] = fgm

    lane = lax.broadcasted_iota(jnp.int32, (1, 128), 1)
    lanef = lane.astype(jnp.float32)

    def pick(i, carry):
        tsc, tx1, ty1, tx2, ty2 = carry
        cur = fgs[...]
        m = jnp.max(cur)
        sel = jnp.min(jnp.where(cur == m, F, 3.0e38))
        fgs[...] = jnp.where(F == sel, NEGINF, cur)
        r = jnp.floor(sel / 128.0)
        ri = r.astype(jnp.int32)
        ci = (sel - r * 128.0).astype(jnp.int32)
        hit = lane == ci

        def gather(ref):
            row = ref[pl.ds(ri, 1), :]
            return jnp.max(jnp.where(hit, row, NEGINF))

        oh = lane == i
        tsc = jnp.where(oh, m, tsc)
        tx1 = jnp.where(oh, gather(x1s), tx1)
        ty1 = jnp.where(oh, gather(y1s), ty1)
        tx2 = jnp.where(oh, gather(x2s), tx2)
        ty2 = jnp.where(oh, gather(y2s), ty2)
        return tsc, tx1, ty1, tx2, ty2

    zeros = jnp.zeros((1, 128), jnp.float32)
    init = (jnp.full((1, 128), NEGINF, jnp.float32), zeros, zeros, zeros, zeros)
    tsc, tx1, ty1, tx2, ty2 = lax.fori_loop(0, PRE, pick, init)

    areas = (tx2 - tx1 + 1.0) * (ty2 - ty1 + 1.0)
    keep0 = jnp.where(lane < PRE, 1.0, 0.0)

    def nms(i, keep):
        selm = lane == i
        x1i = jnp.max(jnp.where(selm, tx1, NEGINF))
        y1i = jnp.max(jnp.where(selm, ty1, NEGINF))
        x2i = jnp.max(jnp.where(selm, tx2, NEGINF))
        y2i = jnp.max(jnp.where(selm, ty2, NEGINF))
        ai = jnp.max(jnp.where(selm, areas, NEGINF))
        ki = jnp.max(jnp.where(selm, keep, 0.0))
        xx1 = jnp.maximum(x1i, tx1)
        yy1 = jnp.maximum(y1i, ty1)
        xx2 = jnp.minimum(x2i, tx2)
        yy2 = jnp.minimum(y2i, ty2)
        iw = jnp.maximum(xx2 - xx1 + 1.0, 0.0)
        ih = jnp.maximum(yy2 - yy1 + 1.0, 0.0)
        inter = iw * ih
        iou = inter / (ai + areas - inter)
        sup = (iou > NMS_T) & (lane > i) & (ki > 0.0)
        return jnp.where(sup, 0.0, keep)

    keep = lax.fori_loop(0, PRE, nms, keep0)

    fsc = jnp.where(keep > 0.0, tsc, NEG)
    fsc = jnp.where(lane < PRE, fsc, NEGINF)

    rowio = lax.broadcasted_iota(jnp.int32, (16, 128), 0)
    colio = lax.broadcasted_iota(jnp.int32, (16, 128), 1)

    def outb(j, carry):
        fs, R = carry
        m = jnp.max(fs)
        sl = jnp.min(jnp.where(fs == m, lanef, 3.0e38))
        slm = lanef == sl
        bx1 = jnp.max(jnp.where(slm, tx1, NEGINF))
        by1 = jnp.max(jnp.where(slm, ty1, NEGINF))
        bx2 = jnp.max(jnp.where(slm, tx2, NEGINF))
        by2 = jnp.max(jnp.where(slm, ty2, NEGINF))
        rj = rowio == j
        R = jnp.where(rj & (colio == 0), bx1, R)
        R = jnp.where(rj & (colio == 1), by1, R)
        R = jnp.where(rj & (colio == 2), bx2, R)
        R = jnp.where(rj & (colio == 3), by2, R)
        fs = jnp.where(slm, NEGINF, fs)
        return fs, R

    _, R = lax.fori_loop(0, POST, outb, (fsc, jnp.zeros((16, 128), jnp.float32)))
    out_ref[...] = R[:POST, :4]


def _flatpad(v):
    vf = v.T.reshape(-1)
    vf = jnp.concatenate([vf, jnp.zeros((NP - N,), vf.dtype)])
    return vf.reshape(RPAD, 128)


def kernel(scores, bbox_deltas, im_info):
    fg = _flatpad(scores[0, A:].reshape(A, HW))
    bd = bbox_deltas[0].reshape(A, 4, HW)
    dxf = _flatpad(bd[:, 0, :])
    dyf = _flatpad(bd[:, 1, :])
    dwf = _flatpad(bd[:, 2, :])
    dhf = _flatpad(bd[:, 3, :])

    return pl.pallas_call(
        _body,
        out_shape=jax.ShapeDtypeStruct((POST, 4), jnp.float32),
        in_specs=[
            pl.BlockSpec(memory_space=pltpu.SMEM),
            pl.BlockSpec(memory_space=pltpu.VMEM),
            pl.BlockSpec(memory_space=pltpu.VMEM),
            pl.BlockSpec(memory_space=pltpu.VMEM),
            pl.BlockSpec(memory_space=pltpu.VMEM),
            pl.BlockSpec(memory_space=pltpu.VMEM),
        ],
        out_specs=pl.BlockSpec(memory_space=pltpu.VMEM),
        scratch_shapes=[pltpu.VMEM((RPAD, 128), jnp.float32)] * 5,
    )(im_info, fg, dxf, dyf, dwf, dhf)


# speedup vs baseline: 7.7782x; 1.1420x over previous
import jax
import jax.numpy as jnp
from jax import lax
from jax.experimental import pallas as pl
from jax.experimental.pallas import tpu as pltpu

A = 9
H = 50
W = 50
HW = H * W
N = A * HW
PRE = 20
POST = 10
NMS_T = 0.7
NEG = -1e9
NEGINF = -3.0e38


def _body(im_ref, fg_ref, dx_ref, dy_ref, dw_ref, dh_ref, out_ref,
          x1s, y1s, x2s, y2s, fgs):
    shape = (A, HW)
    a = lax.broadcasted_iota(jnp.int32, shape, 0).astype(jnp.float32)
    p = lax.broadcasted_iota(jnp.int32, shape, 1).astype(jnp.float32)
    F = p * 9.0 + a
    hh = jnp.floor(p / 50.0)
    ww = p - hh * 50.0
    ridx = jnp.floor(a / 3.0)
    sidx = a - ridx * 3.0
    wr = jnp.where(ridx == 0.0, 23.0, jnp.where(ridx == 1.0, 16.0, 11.0))
    hr = jnp.where(ridx == 0.0, 12.0, jnp.where(ridx == 1.0, 16.0, 22.0))
    sc = jnp.where(sidx == 0.0, 8.0, jnp.where(sidx == 1.0, 16.0, 32.0))
    wa = wr * sc
    ha = hr * sc
    cx = 8.0 + ww * 16.0
    cy = 8.0 + hh * 16.0

    imh = im_ref[0, 0]
    imw = im_ref[0, 1]
    imsc = im_ref[0, 2]

    dx = dx_ref[...]
    dy = dy_ref[...]
    dw = jnp.clip(dw_ref[...], -10.0, 4.135)
    dh = jnp.clip(dh_ref[...], -10.0, 4.135)
    pcx = dx * wa + cx
    pcy = dy * ha + cy
    pw = jnp.exp(dw) * wa
    ph = jnp.exp(dh) * ha
    x1 = jnp.clip(pcx - 0.5 * pw, 0.0, imw - 1.0)
    x2 = jnp.clip(pcx + 0.5 * pw, 0.0, imw - 1.0)
    y1 = jnp.clip(pcy - 0.5 * ph, 0.0, imh - 1.0)
    y2 = jnp.clip(pcy + 0.5 * ph, 0.0, imh - 1.0)
    ws_ = x2 - x1 + 1.0
    hs_ = y2 - y1 + 1.0
    minsz = 5.0 * imsc
    valid = (ws_ >= minsz) & (hs_ >= minsz)
    fgm = jnp.where(valid, fg_ref[...], NEG)

    x1s[...] = x1
    y1s[...] = y1
    x2s[...] = x2
    y2s[...] = y2
    fgs[...] = fgm

    lane = lax.broadcasted_iota(jnp.int32, (1, 128), 1)
    lanef = lane.astype(jnp.float32)
    lhw = lax.broadcasted_iota(jnp.int32, (1, HW), 1)

    def pick(i, carry):
        tsc, tx1, ty1, tx2, ty2 = carry
        cur = fgs[...]
        m = jnp.max(cur)
        sel = jnp.min(jnp.where(cur == m, F, 3.0e38))
        fgs[...] = jnp.where(F == sel, NEGINF, cur)
        pf = jnp.floor(sel / 9.0)
        ai = (sel - pf * 9.0).astype(jnp.int32)
        hit = lhw == pf.astype(jnp.int32)

        def gather(ref):
            row = ref[pl.ds(ai, 1), :]
            return jnp.max(jnp.where(hit, row, NEGINF))

        oh = lane == i
        tsc = jnp.where(oh, m, tsc)
        tx1 = jnp.where(oh, gather(x1s), tx1)
        ty1 = jnp.where(oh, gather(y1s), ty1)
        tx2 = jnp.where(oh, gather(x2s), tx2)
        ty2 = jnp.where(oh, gather(y2s), ty2)
        return tsc, tx1, ty1, tx2, ty2

    zeros = jnp.zeros((1, 128), jnp.float32)
    init = (jnp.full((1, 128), NEGINF, jnp.float32), zeros, zeros, zeros, zeros)
    tsc, tx1, ty1, tx2, ty2 = lax.fori_loop(0, PRE, pick, init)

    areas = (tx2 - tx1 + 1.0) * (ty2 - ty1 + 1.0)
    keep0 = jnp.where(lane < PRE, 1.0, 0.0)

    def nms(i, keep):
        selm = lane == i
        x1i = jnp.max(jnp.where(selm, tx1, NEGINF))
        y1i = jnp.max(jnp.where(selm, ty1, NEGINF))
        x2i = jnp.max(jnp.where(selm, tx2, NEGINF))
        y2i = jnp.max(jnp.where(selm, ty2, NEGINF))
        ai = jnp.max(jnp.where(selm, areas, NEGINF))
        ki = jnp.max(jnp.where(selm, keep, 0.0))
        xx1 = jnp.maximum(x1i, tx1)
        yy1 = jnp.maximum(y1i, ty1)
        xx2 = jnp.minimum(x2i, tx2)
        yy2 = jnp.minimum(y2i, ty2)
        iw = jnp.maximum(xx2 - xx1 + 1.0, 0.0)
        ih = jnp.maximum(yy2 - yy1 + 1.0, 0.0)
        inter = iw * ih
        iou = inter / (ai + areas - inter)
        sup = (iou > NMS_T) & (lane > i) & (ki > 0.0)
        return jnp.where(sup, 0.0, keep)

    keep = lax.fori_loop(0, PRE, nms, keep0)

    fsc = jnp.where(keep > 0.0, tsc, NEG)
    fsc = jnp.where(lane < PRE, fsc, NEGINF)

    rowio = lax.broadcasted_iota(jnp.int32, (16, 128), 0)
    colio = lax.broadcasted_iota(jnp.int32, (16, 128), 1)

    def outb(j, carry):
        fs, R = carry
        m = jnp.max(fs)
        sl = jnp.min(jnp.where(fs == m, lanef, 3.0e38))
        slm = lanef == sl
        bx1 = jnp.max(jnp.where(slm, tx1, NEGINF))
        by1 = jnp.max(jnp.where(slm, ty1, NEGINF))
        bx2 = jnp.max(jnp.where(slm, tx2, NEGINF))
        by2 = jnp.max(jnp.where(slm, ty2, NEGINF))
        rj = rowio == j
        R = jnp.where(rj & (colio == 0), bx1, R)
        R = jnp.where(rj & (colio == 1), by1, R)
        R = jnp.where(rj & (colio == 2), bx2, R)
        R = jnp.where(rj & (colio == 3), by2, R)
        fs = jnp.where(slm, NEGINF, fs)
        return fs, R

    _, R = lax.fori_loop(0, POST, outb, (fsc, jnp.zeros((16, 128), jnp.float32)))
    out_ref[...] = R[:POST, :4]


def kernel(scores, bbox_deltas, im_info):
    fg = scores[0, A:].reshape(A, HW)
    bd = bbox_deltas[0].reshape(A, 4, HW)

    return pl.pallas_call(
        _body,
        out_shape=jax.ShapeDtypeStruct((POST, 4), jnp.float32),
        in_specs=[
            pl.BlockSpec(memory_space=pltpu.SMEM),
            pl.BlockSpec(memory_space=pltpu.VMEM),
            pl.BlockSpec(memory_space=pltpu.VMEM),
            pl.BlockSpec(memory_space=pltpu.VMEM),
            pl.BlockSpec(memory_space=pltpu.VMEM),
            pl.BlockSpec(memory_space=pltpu.VMEM),
        ],
        out_specs=pl.BlockSpec(memory_space=pltpu.VMEM),
        scratch_shapes=[pltpu.VMEM((A, HW), jnp.float32)] * 5,
    )(im_info, fg, bd[:, 0, :], bd[:, 1, :], bd[:, 2, :], bd[:, 3, :])
